# baseline (speedup 1.0000x reference)
import functools

import jax
import jax.numpy as jnp
from jax import lax
from jax.experimental import pallas as pl
from jax.experimental.pallas import tpu as pltpu
from jax.experimental.pallas import tpu_sc as plsc

N_EXPERTS = 64
TOP_K = 8
BLOCK_ROWS = 512
N_TOKENS = 16384
NW = 32
TPW = N_TOKENS // NW
L = 16


def _logits_body(x_ref, w_ref, lt_ref):
    w = w_ref[...]
    logits = jax.lax.dot_general(
        x_ref[...], w, (((1,), (1,)), ((), ())),
        preferred_element_type=jnp.float32,
    )
    lt_ref[...] = logits.T.reshape(1, N_EXPERTS, BLOCK_ROWS)


def _routing_body(lt_hbm, idx_hbm, wgt_hbm, lt_v, idx_v, wgt_v):
    wid = lax.axis_index("s") * 2 + lax.axis_index("c")
    pltpu.sync_copy(lt_hbm.at[wid], lt_v)

    def group(g, _):
        base = g * L
        m = jnp.full((L,), -jnp.inf, dtype=jnp.float32)
        for e in range(N_EXPERTS):
            m = jnp.maximum(m, lt_v[e, pl.ds(base, L)])
        s = jnp.zeros((L,), dtype=jnp.float32)
        for e in range(N_EXPERTS):
            s = s + jnp.exp(lt_v[e, pl.ds(base, L)] - m)
        rs = 1.0 / s
        tok = base + lax.iota(jnp.int32, L)
        for k in range(TOP_K):
            mk = jnp.full((L,), -jnp.inf, dtype=jnp.float32)
            ak = jnp.zeros((L,), dtype=jnp.int32)
            for e in range(N_EXPERTS):
                v = lt_v[e, pl.ds(base, L)]
                upd = v > mk
                mk = jnp.where(upd, v, mk)
                ak = jnp.where(upd, e, ak)
            idx_v[k, pl.ds(base, L)] = ak
            wgt_v[k, pl.ds(base, L)] = jnp.exp(mk - m) * rs
            plsc.store_scatter(
                lt_v, [ak, tok], jnp.full((L,), -jnp.inf, dtype=jnp.float32)
            )
        return 0

    lax.fori_loop(0, TPW // L, group, 0)
    pltpu.sync_copy(idx_v, idx_hbm.at[wid])
    pltpu.sync_copy(wgt_v, wgt_hbm.at[wid])


@jax.jit
def kernel(hidden_states, weight):
    bsz, seq_len, h = hidden_states.shape
    n = bsz * seq_len
    x = hidden_states.reshape(n, h)
    grid = (n // BLOCK_ROWS,)
    lt = pl.pallas_call(
        _logits_body,
        grid=grid,
        in_specs=[
            pl.BlockSpec((BLOCK_ROWS, h), lambda i: (i, 0)),
            pl.BlockSpec((N_EXPERTS, h), lambda i: (0, 0)),
        ],
        out_specs=pl.BlockSpec((1, N_EXPERTS, BLOCK_ROWS), lambda i: (i, 0, 0)),
        out_shape=jax.ShapeDtypeStruct((NW, N_EXPERTS, TPW), jnp.float32),
        compiler_params=pltpu.CompilerParams(
            dimension_semantics=("parallel",),
        ),
    )(x, weight)

    routing = functools.partial(
        pl.kernel,
        out_type=[
            jax.ShapeDtypeStruct((NW, TOP_K, TPW), jnp.int32),
            jax.ShapeDtypeStruct((NW, TOP_K, TPW), jnp.float32),
        ],
        mesh=plsc.VectorSubcoreMesh(core_axis_name="c", subcore_axis_name="s"),
        compiler_params=pltpu.CompilerParams(needs_layout_passes=False),
        scratch_types=[
            pltpu.VMEM((N_EXPERTS, TPW), jnp.float32),
            pltpu.VMEM((TOP_K, TPW), jnp.int32),
            pltpu.VMEM((TOP_K, TPW), jnp.float32),
        ],
    )(_routing_body)
    idx3, wgt3 = routing(lt)
    idx = idx3.transpose(0, 2, 1).reshape(n, TOP_K)
    wgt = wgt3.transpose(0, 2, 1).reshape(n, TOP_K)
    return idx, wgt


# speedup vs baseline: 1.5177x; 1.5177x over previous
import jax
import jax.numpy as jnp
from jax.experimental import pallas as pl
from jax.experimental.pallas import tpu as pltpu

N_EXPERTS = 64
TOP_K = 8
BLOCK_ROWS = 1024


def _gate_body(x_ref, w_ref, idx_ref, wgt_ref):
    w = w_ref[...]
    logits = jax.lax.dot_general(
        x_ref[...], w, (((1,), (1,)), ((), ())),
        preferred_element_type=jnp.float32,
    )
    lt = logits.T
    m = jnp.max(lt, axis=0, keepdims=True)
    e = jnp.exp(lt - m)
    rs = 1.0 / jnp.sum(e, axis=0, keepdims=True)

    row = jax.lax.broadcasted_iota(jnp.int32, lt.shape, 0).astype(jnp.float32)
    neg_inf = jnp.float32(-jnp.inf)
    idxs = []
    vals = []
    sc = lt
    for _ in range(TOP_K):
        mk = jnp.max(sc, axis=0, keepdims=True)
        ak = jnp.min(
            jnp.where(sc == mk, row, jnp.float32(N_EXPERTS)), axis=0, keepdims=True
        )
        vals.append(jnp.exp(mk - m) * rs)
        idxs.append(ak)
        sc = jnp.where(row == ak, neg_inf, sc)
    idx_ref[...] = jnp.concatenate(idxs, axis=0).astype(jnp.int32)
    wgt_ref[...] = jnp.concatenate(vals, axis=0)


@jax.jit
def kernel(hidden_states, weight):
    bsz, seq_len, h = hidden_states.shape
    n = bsz * seq_len
    x = hidden_states.reshape(n, h)
    grid = (n // BLOCK_ROWS,)
    idx_t, wgt_t = pl.pallas_call(
        _gate_body,
        grid=grid,
        in_specs=[
            pl.BlockSpec((BLOCK_ROWS, h), lambda i: (i, 0)),
            pl.BlockSpec((N_EXPERTS, h), lambda i: (0, 0)),
        ],
        out_specs=[
            pl.BlockSpec((TOP_K, BLOCK_ROWS), lambda i: (0, i)),
            pl.BlockSpec((TOP_K, BLOCK_ROWS), lambda i: (0, i)),
        ],
        out_shape=[
            jax.ShapeDtypeStruct((TOP_K, n), jnp.int32),
            jax.ShapeDtypeStruct((TOP_K, n), jnp.float32),
        ],
        compiler_params=pltpu.CompilerParams(
            dimension_semantics=("parallel",),
        ),
    )(x, weight)
    return idx_t.T, wgt_t.T
